# Initial kernel scaffold; baseline (speedup 1.0000x reference)
import jax
import jax.numpy as jnp
from jax.experimental import pallas as pl


def kernel(x, tape_init_re, tape_init_im, eta_raw, alpha, epsilon_factor, epsilon_scale, epsilon_diag, pred_factor, pred_scale, pred_diag, torque_rotation, w_r, breadth_gate, basis):
    raise NotImplementedError("write your pallas kernel here")


# speedup vs baseline: 11.4159x; 11.4159x over previous
import functools

import jax
import jax.numpy as jnp
from jax.experimental import pallas as pl
from jax.experimental.pallas import tpu as pltpu

HIDDEN_DIM = 1024
MEMORY_DIM = 1024
TOTAL_SLOTS = 1040
GAMMA = 0.92


def _weights_kernel(basis_ref, efac_ref, escale_ref, ediag_ref,
                    pfac_ref, pscale_ref, pdiag_ref,
                    breadth_ref, torque_ref, wr_ref, eta_ref,
                    wa_ref, wc_ref, crot_ref, srot_ref):
    basis = basis_ref[...]
    efac = efac_ref[...]
    pfac = pfac_ref[...]
    breadth = 1.0 + jnp.tanh(breadth_ref[...])
    eta = jax.nn.softplus(eta_ref[0, 0])

    b1 = basis[:, :MEMORY_DIM]
    br1 = breadth[:, :MEMORY_DIM]
    ed1 = ediag_ref[...][:, :MEMORY_DIM]

    f = jnp.dot(basis, breadth.T * efac,
                preferred_element_type=jnp.float32)
    low = jnp.dot(f * escale_ref[...],
                  efac[:MEMORY_DIM, :].T,
                  preferred_element_type=jnp.float32)
    wa_ref[...] = eta * (b1 * ((1.0 + ed1) * br1) + low)

    bp = jnp.dot(basis, pfac, preferred_element_type=jnp.float32)
    pf1 = pfac[:MEMORY_DIM, :]
    wc_ref[...] = (jnp.dot(pf1 * pscale_ref[...], bp.T,
                           preferred_element_type=jnp.float32)
                   + pdiag_ref[...][:, :MEMORY_DIM].T * b1.T)

    leak = jax.nn.sigmoid(wr_ref[...][:, :MEMORY_DIM])
    tq = torque_ref[...][:, :MEMORY_DIM]
    g = GAMMA * leak
    crot_ref[...] = (g * jnp.cos(tq)).reshape(8, 128)
    srot_ref[...] = (g * jnp.sin(tq)).reshape(8, 128)


def _fold_weights(basis, efac, escale, ediag, pfac, pscale, pdiag,
                  breadth_gate, torque, w_r, eta_raw, interpret=False):
    out_shapes = (
        jax.ShapeDtypeStruct((MEMORY_DIM, MEMORY_DIM), jnp.float32),
        jax.ShapeDtypeStruct((MEMORY_DIM, MEMORY_DIM), jnp.float32),
        jax.ShapeDtypeStruct((8, 128), jnp.float32),
        jax.ShapeDtypeStruct((8, 128), jnp.float32),
    )
    return pl.pallas_call(
        _weights_kernel,
        out_shape=out_shapes,
        interpret=interpret,
    )(basis, efac, escale.reshape(1, -1), ediag.reshape(1, -1),
      pfac, pscale.reshape(1, -1), pdiag.reshape(1, -1),
      breadth_gate.reshape(1, -1), torque.reshape(1, -1),
      w_r.reshape(1, -1), eta_raw.reshape(1, 1))


def _matmul_kernel(x_ref, w_ref, o_ref):
    o_ref[...] = jnp.dot(x_ref[...], w_ref[...],
                         preferred_element_type=jnp.float32)


def _matmul_add_kernel(x_ref, w_ref, r_ref, alpha_ref, o_ref):
    o_ref[...] = (jnp.dot(x_ref[...], w_ref[...],
                          preferred_element_type=jnp.float32)
                  + alpha_ref[0] * r_ref[...])


def _scan_kernel(crot_ref, srot_ref, init_re_ref, init_im_ref, drive_ref,
                 nre_ref, tre_s, tim_s, *, tb):
    ch = pl.program_id(1)

    @pl.when(ch == 0)
    def _():
        tre_s[...] = init_re_ref[...]
        tim_s[...] = init_im_ref[...]

    cr = crot_ref[...]
    sr = srot_ref[...]

    def body(i, carry):
        tre, tim = carry
        d = drive_ref[0, i]
        ure = cr * tre - sr * tim + d
        uim = sr * tre + cr * tim
        s = jnp.sum(ure * ure) + jnp.sum(uim * uim)
        inv = 1.0 / jnp.maximum(jnp.sqrt(s), 1e-8)
        nre = ure * inv
        nim = uim * inv
        nre_ref[0, i] = nre
        return nre, nim

    tre, tim = jax.lax.fori_loop(0, tb, body, (tre_s[...], tim_s[...]))
    tre_s[...] = tre
    tim_s[...] = tim


def _run_scan(crot, srot, init_re, init_im, drive4, *, tb, interpret=False):
    b, t = drive4.shape[0], drive4.shape[1]
    nch = t // tb
    grid = (b, nch)
    kern = functools.partial(_scan_kernel, tb=tb)
    return pl.pallas_call(
        kern,
        grid=grid,
        in_specs=[
            pl.BlockSpec((8, 128), lambda bb, cc: (0, 0)),
            pl.BlockSpec((8, 128), lambda bb, cc: (0, 0)),
            pl.BlockSpec((8, 128), lambda bb, cc: (0, 0)),
            pl.BlockSpec((8, 128), lambda bb, cc: (0, 0)),
            pl.BlockSpec((1, tb, 8, 128), lambda bb, cc: (bb, cc, 0, 0)),
        ],
        out_specs=pl.BlockSpec((1, tb, 8, 128), lambda bb, cc: (bb, cc, 0, 0)),
        out_shape=jax.ShapeDtypeStruct((b, t, 8, 128), jnp.float32),
        scratch_shapes=[pltpu.VMEM((8, 128), jnp.float32),
                        pltpu.VMEM((8, 128), jnp.float32)],
        compiler_params=pltpu.CompilerParams(
            dimension_semantics=("arbitrary", "arbitrary")),
        interpret=interpret,
    )(crot, srot, init_re, init_im, drive4)


def _matmul(x, w, *, bm, interpret=False):
    m, k = x.shape
    n = w.shape[1]
    return pl.pallas_call(
        _matmul_kernel,
        grid=(m // bm,),
        in_specs=[pl.BlockSpec((bm, k), lambda i: (i, 0)),
                  pl.BlockSpec((k, n), lambda i: (0, 0))],
        out_specs=pl.BlockSpec((bm, n), lambda i: (i, 0)),
        out_shape=jax.ShapeDtypeStruct((m, n), jnp.float32),
        compiler_params=pltpu.CompilerParams(
            dimension_semantics=("arbitrary",)),
        interpret=interpret,
    )(x, w)


def _matmul_add(x, w, r, alpha, *, bm, interpret=False):
    m, k = x.shape
    n = w.shape[1]
    return pl.pallas_call(
        _matmul_add_kernel,
        grid=(m // bm,),
        in_specs=[pl.BlockSpec((bm, k), lambda i: (i, 0)),
                  pl.BlockSpec((k, n), lambda i: (0, 0)),
                  pl.BlockSpec((bm, n), lambda i: (i, 0)),
                  pl.BlockSpec(memory_space=pltpu.SMEM)],
        out_specs=pl.BlockSpec((bm, n), lambda i: (i, 0)),
        out_shape=jax.ShapeDtypeStruct((m, n), jnp.float32),
        compiler_params=pltpu.CompilerParams(
            dimension_semantics=("arbitrary",)),
        interpret=interpret,
    )(x, w, r, alpha.reshape(1))


def _kernel_impl(x, tape_init_re, tape_init_im, eta_raw, alpha,
                 epsilon_factor, epsilon_scale, epsilon_diag,
                 pred_factor, pred_scale, pred_diag,
                 torque_rotation, w_r, breadth_gate, basis,
                 interpret=False):
    b, t, h = x.shape
    wa, wc, crot, srot = _fold_weights(
        basis, epsilon_factor, epsilon_scale, epsilon_diag,
        pred_factor, pred_scale, pred_diag,
        breadth_gate, torque_rotation, w_r, eta_raw, interpret=interpret)

    x2 = x.reshape(b * t, h)
    drive = _matmul(x2, wa, bm=512, interpret=interpret)
    drive4 = drive.reshape(b, t, 8, 128)

    init_re = tape_init_re[:MEMORY_DIM].reshape(8, 128)
    init_im = tape_init_im[:MEMORY_DIM].reshape(8, 128)
    nre4 = _run_scan(crot, srot, init_re, init_im, drive4,
                     tb=256, interpret=interpret)

    nre2 = nre4.reshape(b * t, MEMORY_DIM)
    y2 = _matmul_add(nre2, wc, x2, alpha, bm=512, interpret=interpret)
    return y2.reshape(b, t, h)


def kernel(x, tape_init_re, tape_init_im, eta_raw, alpha,
           epsilon_factor, epsilon_scale, epsilon_diag,
           pred_factor, pred_scale, pred_diag,
           torque_rotation, w_r, breadth_gate, basis):
    return _kernel_impl(x, tape_init_re, tape_init_im, eta_raw, alpha,
                        epsilon_factor, epsilon_scale, epsilon_diag,
                        pred_factor, pred_scale, pred_diag,
                        torque_rotation, w_r, breadth_gate, basis)


# speedup vs baseline: 34.7799x; 3.0466x over previous
import functools

import jax
import jax.numpy as jnp
from jax.experimental import pallas as pl
from jax.experimental.pallas import tpu as pltpu

HIDDEN_DIM = 1024
MEMORY_DIM = 1024
TOTAL_SLOTS = 1040
GAMMA = 0.92


def _weights_kernel(basis_ref, efac_ref, escale_ref, ediag_ref,
                    pfac_ref, pscale_ref, pdiag_ref,
                    breadth_ref, torque_ref, wr_ref, eta_ref,
                    wa_ref, wc_ref, crot_ref, srot_ref):
    basis = basis_ref[...]
    efac = efac_ref[...]
    pfac = pfac_ref[...]
    breadth = 1.0 + jnp.tanh(breadth_ref[...])
    eta = jax.nn.softplus(eta_ref[0, 0])

    b1 = basis[:, :MEMORY_DIM]
    br1 = breadth[:, :MEMORY_DIM]
    ed1 = ediag_ref[...][:, :MEMORY_DIM]

    f = jnp.dot(basis, breadth.T * efac,
                preferred_element_type=jnp.float32)
    low = jnp.dot(f * escale_ref[...],
                  efac[:MEMORY_DIM, :].T,
                  preferred_element_type=jnp.float32)
    wa_ref[...] = eta * (b1 * ((1.0 + ed1) * br1) + low)

    bp = jnp.dot(basis, pfac, preferred_element_type=jnp.float32)
    pf1 = pfac[:MEMORY_DIM, :]
    wc_ref[...] = (jnp.dot(pf1 * pscale_ref[...], bp.T,
                           preferred_element_type=jnp.float32)
                   + pdiag_ref[...][:, :MEMORY_DIM].T * b1.T)

    leak = jax.nn.sigmoid(wr_ref[...][:, :MEMORY_DIM])
    tq = torque_ref[...][:, :MEMORY_DIM]
    g = GAMMA * leak
    crot_ref[...] = (g * jnp.cos(tq)).reshape(8, 128)
    srot_ref[...] = (g * jnp.sin(tq)).reshape(8, 128)


def _fold_weights(basis, efac, escale, ediag, pfac, pscale, pdiag,
                  breadth_gate, torque, w_r, eta_raw, interpret=False):
    out_shapes = (
        jax.ShapeDtypeStruct((MEMORY_DIM, MEMORY_DIM), jnp.float32),
        jax.ShapeDtypeStruct((MEMORY_DIM, MEMORY_DIM), jnp.float32),
        jax.ShapeDtypeStruct((8, 128), jnp.float32),
        jax.ShapeDtypeStruct((8, 128), jnp.float32),
    )
    return pl.pallas_call(
        _weights_kernel,
        out_shape=out_shapes,
        interpret=interpret,
    )(basis, efac, escale.reshape(1, -1), ediag.reshape(1, -1),
      pfac, pscale.reshape(1, -1), pdiag.reshape(1, -1),
      breadth_gate.reshape(1, -1), torque.reshape(1, -1),
      w_r.reshape(1, -1), eta_raw.reshape(1, 1))


def _matmul_kernel(x_ref, w_ref, o_ref):
    o_ref[...] = jnp.dot(x_ref[...], w_ref[...],
                         preferred_element_type=jnp.float32)


def _matmul_add_kernel(x_ref, w_ref, r_ref, alpha_ref, o_ref):
    o_ref[...] = (jnp.dot(x_ref[...], w_ref[...],
                          preferred_element_type=jnp.float32)
                  + alpha_ref[0] * r_ref[...])


def _scan_kernel(crot_ref, srot_ref, init_re_ref, init_im_ref, drive_ref,
                 nre_ref, tre_s, tim_s, *, tb, nb):
    ch = pl.program_id(0)

    @pl.when(ch == 0)
    def _():
        tre_s[...] = jnp.broadcast_to(init_re_ref[...], (nb, 8, 128))
        tim_s[...] = jnp.broadcast_to(init_im_ref[...], (nb, 8, 128))

    cr = crot_ref[...]
    sr = srot_ref[...]

    def body(i, carry):
        tre, tim = carry
        d = drive_ref[:, i]
        ure = cr * tre - sr * tim + d
        uim = sr * tre + cr * tim
        s = jnp.sum(ure * ure + uim * uim, axis=(1, 2), keepdims=True)
        inv = jnp.minimum(jax.lax.rsqrt(s), 1e8)
        nre = ure * inv
        nim = uim * inv
        nre_ref[:, i] = nre
        return nre, nim

    tre, tim = jax.lax.fori_loop(0, tb, body, (tre_s[...], tim_s[...]))
    tre_s[...] = tre
    tim_s[...] = tim


def _run_scan(crot, srot, init_re, init_im, drive4, *, tb, interpret=False):
    b, t = drive4.shape[0], drive4.shape[1]
    nch = t // tb
    grid = (nch,)
    kern = functools.partial(_scan_kernel, tb=tb, nb=b)
    return pl.pallas_call(
        kern,
        grid=grid,
        in_specs=[
            pl.BlockSpec((8, 128), lambda cc: (0, 0)),
            pl.BlockSpec((8, 128), lambda cc: (0, 0)),
            pl.BlockSpec((8, 128), lambda cc: (0, 0)),
            pl.BlockSpec((8, 128), lambda cc: (0, 0)),
            pl.BlockSpec((b, tb, 8, 128), lambda cc: (0, cc, 0, 0)),
        ],
        out_specs=pl.BlockSpec((b, tb, 8, 128), lambda cc: (0, cc, 0, 0)),
        out_shape=jax.ShapeDtypeStruct((b, t, 8, 128), jnp.float32),
        scratch_shapes=[pltpu.VMEM((b, 8, 128), jnp.float32),
                        pltpu.VMEM((b, 8, 128), jnp.float32)],
        compiler_params=pltpu.CompilerParams(
            dimension_semantics=("arbitrary",)),
        interpret=interpret,
    )(crot, srot, init_re, init_im, drive4)


def _matmul(x, w, *, bm, interpret=False):
    m, k = x.shape
    n = w.shape[1]
    return pl.pallas_call(
        _matmul_kernel,
        grid=(m // bm,),
        in_specs=[pl.BlockSpec((bm, k), lambda i: (i, 0)),
                  pl.BlockSpec((k, n), lambda i: (0, 0))],
        out_specs=pl.BlockSpec((bm, n), lambda i: (i, 0)),
        out_shape=jax.ShapeDtypeStruct((m, n), jnp.float32),
        compiler_params=pltpu.CompilerParams(
            dimension_semantics=("arbitrary",)),
        interpret=interpret,
    )(x, w)


def _matmul_add(x, w, r, alpha, *, bm, interpret=False):
    m, k = x.shape
    n = w.shape[1]
    return pl.pallas_call(
        _matmul_add_kernel,
        grid=(m // bm,),
        in_specs=[pl.BlockSpec((bm, k), lambda i: (i, 0)),
                  pl.BlockSpec((k, n), lambda i: (0, 0)),
                  pl.BlockSpec((bm, n), lambda i: (i, 0)),
                  pl.BlockSpec(memory_space=pltpu.SMEM)],
        out_specs=pl.BlockSpec((bm, n), lambda i: (i, 0)),
        out_shape=jax.ShapeDtypeStruct((m, n), jnp.float32),
        compiler_params=pltpu.CompilerParams(
            dimension_semantics=("arbitrary",)),
        interpret=interpret,
    )(x, w, r, alpha.reshape(1))


def _kernel_impl(x, tape_init_re, tape_init_im, eta_raw, alpha,
                 epsilon_factor, epsilon_scale, epsilon_diag,
                 pred_factor, pred_scale, pred_diag,
                 torque_rotation, w_r, breadth_gate, basis,
                 interpret=False):
    b, t, h = x.shape
    wa, wc, crot, srot = _fold_weights(
        basis, epsilon_factor, epsilon_scale, epsilon_diag,
        pred_factor, pred_scale, pred_diag,
        breadth_gate, torque_rotation, w_r, eta_raw, interpret=interpret)

    x2 = x.reshape(b * t, h)
    drive = _matmul(x2, wa, bm=512, interpret=interpret)
    drive4 = drive.reshape(b, t, 8, 128)

    init_re = tape_init_re[:MEMORY_DIM].reshape(8, 128)
    init_im = tape_init_im[:MEMORY_DIM].reshape(8, 128)
    nre4 = _run_scan(crot, srot, init_re, init_im, drive4,
                     tb=256, interpret=interpret)

    nre2 = nre4.reshape(b * t, MEMORY_DIM)
    y2 = _matmul_add(nre2, wc, x2, alpha, bm=512, interpret=interpret)
    return y2.reshape(b, t, h)


def kernel(x, tape_init_re, tape_init_im, eta_raw, alpha,
           epsilon_factor, epsilon_scale, epsilon_diag,
           pred_factor, pred_scale, pred_diag,
           torque_rotation, w_r, breadth_gate, basis):
    return _kernel_impl(x, tape_init_re, tape_init_im, eta_raw, alpha,
                        epsilon_factor, epsilon_scale, epsilon_diag,
                        pred_factor, pred_scale, pred_diag,
                        torque_rotation, w_r, breadth_gate, basis)


# speedup vs baseline: 40.4346x; 1.1626x over previous
import functools

import jax
import jax.numpy as jnp
from jax.experimental import pallas as pl
from jax.experimental.pallas import tpu as pltpu

HIDDEN_DIM = 1024
MEMORY_DIM = 1024
TOTAL_SLOTS = 1040
GAMMA = 0.92


def _weights_kernel(basis_ref, efac_ref, escale_ref, ediag_ref,
                    pfac_ref, pscale_ref, pdiag_ref,
                    breadth_ref, torque_ref, wr_ref, eta_ref,
                    wa_ref, wc_ref, crot_ref, srot_ref):
    basis = basis_ref[...]
    efac = efac_ref[...]
    pfac = pfac_ref[...]
    breadth = 1.0 + jnp.tanh(breadth_ref[...])
    eta = jax.nn.softplus(eta_ref[0, 0])

    b1 = basis[:, :MEMORY_DIM]
    br1 = breadth[:, :MEMORY_DIM]
    ed1 = ediag_ref[...][:, :MEMORY_DIM]

    f = jnp.dot(basis, breadth.T * efac,
                preferred_element_type=jnp.float32)
    low = jnp.dot(f * escale_ref[...],
                  efac[:MEMORY_DIM, :].T,
                  preferred_element_type=jnp.float32)
    wa_ref[...] = eta * (b1 * ((1.0 + ed1) * br1) + low)

    bp = jnp.dot(basis, pfac, preferred_element_type=jnp.float32)
    pf1 = pfac[:MEMORY_DIM, :]
    wc_ref[...] = (jnp.dot(pf1 * pscale_ref[...], bp.T,
                           preferred_element_type=jnp.float32)
                   + pdiag_ref[...][:, :MEMORY_DIM].T * b1.T)

    leak = jax.nn.sigmoid(wr_ref[...][:, :MEMORY_DIM])
    tq = torque_ref[...][:, :MEMORY_DIM]
    g = GAMMA * leak
    crot_ref[...] = (g * jnp.cos(tq)).reshape(8, 128)
    srot_ref[...] = (g * jnp.sin(tq)).reshape(8, 128)


def _fold_weights(basis, efac, escale, ediag, pfac, pscale, pdiag,
                  breadth_gate, torque, w_r, eta_raw, interpret=False):
    out_shapes = (
        jax.ShapeDtypeStruct((MEMORY_DIM, MEMORY_DIM), jnp.float32),
        jax.ShapeDtypeStruct((MEMORY_DIM, MEMORY_DIM), jnp.float32),
        jax.ShapeDtypeStruct((8, 128), jnp.float32),
        jax.ShapeDtypeStruct((8, 128), jnp.float32),
    )
    return pl.pallas_call(
        _weights_kernel,
        out_shape=out_shapes,
        interpret=interpret,
    )(basis, efac, escale.reshape(1, -1), ediag.reshape(1, -1),
      pfac, pscale.reshape(1, -1), pdiag.reshape(1, -1),
      breadth_gate.reshape(1, -1), torque.reshape(1, -1),
      w_r.reshape(1, -1), eta_raw.reshape(1, 1))


def _drive_kernel(x_ref, w_ref, o_ref, *, bm):
    r = jnp.dot(x_ref[0], w_ref[...], preferred_element_type=jnp.float32)
    o_ref[...] = r.reshape(1, bm, 8, 128)


def _drive_matmul(x3, w, *, bm, interpret=False):
    b, t, h = x3.shape
    return pl.pallas_call(
        functools.partial(_drive_kernel, bm=bm),
        grid=(b, t // bm),
        in_specs=[pl.BlockSpec((1, bm, h), lambda i, j: (i, j, 0)),
                  pl.BlockSpec((h, MEMORY_DIM), lambda i, j: (0, 0))],
        out_specs=pl.BlockSpec((1, bm, 8, 128), lambda i, j: (i, j, 0, 0)),
        out_shape=jax.ShapeDtypeStruct((b, t, 8, 128), jnp.float32),
        compiler_params=pltpu.CompilerParams(
            dimension_semantics=("arbitrary", "arbitrary")),
        interpret=interpret,
    )(x3, w)


def _out_kernel(n_ref, w_ref, x_ref, alpha_ref, o_ref, *, bm):
    n2 = n_ref[...].reshape(bm, MEMORY_DIM)
    o_ref[0] = (jnp.dot(n2, w_ref[...], preferred_element_type=jnp.float32)
                + alpha_ref[0] * x_ref[0])


def _out_matmul(nre4, w, x3, alpha, *, bm, interpret=False):
    b, t, h = x3.shape
    return pl.pallas_call(
        functools.partial(_out_kernel, bm=bm),
        grid=(b, t // bm),
        in_specs=[pl.BlockSpec((1, bm, 8, 128), lambda i, j: (i, j, 0, 0)),
                  pl.BlockSpec((MEMORY_DIM, h), lambda i, j: (0, 0)),
                  pl.BlockSpec((1, bm, h), lambda i, j: (i, j, 0)),
                  pl.BlockSpec(memory_space=pltpu.SMEM)],
        out_specs=pl.BlockSpec((1, bm, h), lambda i, j: (i, j, 0)),
        out_shape=jax.ShapeDtypeStruct((b, t, h), jnp.float32),
        compiler_params=pltpu.CompilerParams(
            dimension_semantics=("arbitrary", "arbitrary")),
        interpret=interpret,
    )(nre4, w, x3, alpha.reshape(1))


def _scan_kernel(crot_ref, srot_ref, init_re_ref, init_im_ref, drive_ref,
                 nre_ref, tre_s, tim_s, *, tb, nb):
    ch = pl.program_id(0)

    @pl.when(ch == 0)
    def _():
        tre_s[...] = jnp.broadcast_to(init_re_ref[...], (nb, 8, 128))
        tim_s[...] = jnp.broadcast_to(init_im_ref[...], (nb, 8, 128))

    cr = crot_ref[...]
    sr = srot_ref[...]

    def body(i, carry):
        tre, tim = carry
        d = drive_ref[:, i]
        ure = cr * tre - sr * tim + d
        uim = sr * tre + cr * tim
        s = jnp.sum(ure * ure + uim * uim, axis=(1, 2), keepdims=True)
        inv = jnp.minimum(jax.lax.rsqrt(s), 1e8)
        nre = ure * inv
        nim = uim * inv
        nre_ref[:, i] = nre
        return nre, nim

    tre, tim = jax.lax.fori_loop(0, tb, body, (tre_s[...], tim_s[...]))
    tre_s[...] = tre
    tim_s[...] = tim


def _run_scan(crot, srot, init_re, init_im, drive4, *, tb, interpret=False):
    b, t = drive4.shape[0], drive4.shape[1]
    nch = t // tb
    grid = (nch,)
    kern = functools.partial(_scan_kernel, tb=tb, nb=b)
    return pl.pallas_call(
        kern,
        grid=grid,
        in_specs=[
            pl.BlockSpec((8, 128), lambda cc: (0, 0)),
            pl.BlockSpec((8, 128), lambda cc: (0, 0)),
            pl.BlockSpec((8, 128), lambda cc: (0, 0)),
            pl.BlockSpec((8, 128), lambda cc: (0, 0)),
            pl.BlockSpec((b, tb, 8, 128), lambda cc: (0, cc, 0, 0)),
        ],
        out_specs=pl.BlockSpec((b, tb, 8, 128), lambda cc: (0, cc, 0, 0)),
        out_shape=jax.ShapeDtypeStruct((b, t, 8, 128), jnp.float32),
        scratch_shapes=[pltpu.VMEM((b, 8, 128), jnp.float32),
                        pltpu.VMEM((b, 8, 128), jnp.float32)],
        compiler_params=pltpu.CompilerParams(
            dimension_semantics=("arbitrary",)),
        interpret=interpret,
    )(crot, srot, init_re, init_im, drive4)


def _kernel_impl(x, tape_init_re, tape_init_im, eta_raw, alpha,
                 epsilon_factor, epsilon_scale, epsilon_diag,
                 pred_factor, pred_scale, pred_diag,
                 torque_rotation, w_r, breadth_gate, basis,
                 interpret=False):
    b, t, h = x.shape
    wa, wc, crot, srot = _fold_weights(
        basis, epsilon_factor, epsilon_scale, epsilon_diag,
        pred_factor, pred_scale, pred_diag,
        breadth_gate, torque_rotation, w_r, eta_raw, interpret=interpret)

    drive4 = _drive_matmul(x, wa, bm=512, interpret=interpret)

    init_re = tape_init_re[:MEMORY_DIM].reshape(8, 128)
    init_im = tape_init_im[:MEMORY_DIM].reshape(8, 128)
    nre4 = _run_scan(crot, srot, init_re, init_im, drive4,
                     tb=256, interpret=interpret)

    return _out_matmul(nre4, wc, x, alpha, bm=512, interpret=interpret)


def kernel(x, tape_init_re, tape_init_im, eta_raw, alpha,
           epsilon_factor, epsilon_scale, epsilon_diag,
           pred_factor, pred_scale, pred_diag,
           torque_rotation, w_r, breadth_gate, basis):
    return _kernel_impl(x, tape_init_re, tape_init_im, eta_raw, alpha,
                        epsilon_factor, epsilon_scale, epsilon_diag,
                        pred_factor, pred_scale, pred_diag,
                        torque_rotation, w_r, breadth_gate, basis)


# speedup vs baseline: 68.0447x; 1.6828x over previous
import functools

import jax
import jax.numpy as jnp
from jax.experimental import pallas as pl
from jax.experimental.pallas import tpu as pltpu

HIDDEN_DIM = 1024
MEMORY_DIM = 1024
TOTAL_SLOTS = 1040
GAMMA = 0.92


def _weights_kernel(basis_ref, efac_ref, escale_ref, ediag_ref,
                    pfac_ref, pscale_ref, pdiag_ref,
                    breadth_ref, torque_ref, wr_ref, eta_ref,
                    wa_ref, wc_ref, crot_ref, srot_ref):
    basis = basis_ref[...]
    efac = efac_ref[...]
    pfac = pfac_ref[...]
    breadth = 1.0 + jnp.tanh(breadth_ref[...])
    eta = jax.nn.softplus(eta_ref[0, 0])

    b1 = basis[:, :MEMORY_DIM]
    br1 = breadth[:, :MEMORY_DIM]
    ed1 = ediag_ref[...][:, :MEMORY_DIM]

    f = jnp.dot(basis, breadth.T * efac,
                preferred_element_type=jnp.float32)
    low = jnp.dot(f * escale_ref[...],
                  efac[:MEMORY_DIM, :].T,
                  preferred_element_type=jnp.float32)
    wa_ref[...] = eta * (b1 * ((1.0 + ed1) * br1) + low)

    bp = jnp.dot(basis, pfac, preferred_element_type=jnp.float32)
    pf1 = pfac[:MEMORY_DIM, :]
    wc_ref[...] = (jnp.dot(pf1 * pscale_ref[...], bp.T,
                           preferred_element_type=jnp.float32)
                   + pdiag_ref[...][:, :MEMORY_DIM].T * b1.T)

    leak = jax.nn.sigmoid(wr_ref[...][:, :MEMORY_DIM])
    tq = torque_ref[...][:, :MEMORY_DIM]
    g = GAMMA * leak
    crot_ref[...] = (g * jnp.cos(tq)).reshape(8, 128)
    srot_ref[...] = (g * jnp.sin(tq)).reshape(8, 128)


def _fold_weights(basis, efac, escale, ediag, pfac, pscale, pdiag,
                  breadth_gate, torque, w_r, eta_raw, interpret=False):
    out_shapes = (
        jax.ShapeDtypeStruct((MEMORY_DIM, MEMORY_DIM), jnp.float32),
        jax.ShapeDtypeStruct((MEMORY_DIM, MEMORY_DIM), jnp.float32),
        jax.ShapeDtypeStruct((8, 128), jnp.float32),
        jax.ShapeDtypeStruct((8, 128), jnp.float32),
    )
    return pl.pallas_call(
        _weights_kernel,
        out_shape=out_shapes,
        interpret=interpret,
    )(basis, efac, escale.reshape(1, -1), ediag.reshape(1, -1),
      pfac, pscale.reshape(1, -1), pdiag.reshape(1, -1),
      breadth_gate.reshape(1, -1), torque.reshape(1, -1),
      w_r.reshape(1, -1), eta_raw.reshape(1, 1))


def _drive_kernel(x_ref, w_ref, o_ref, *, bm):
    r = jnp.dot(x_ref[0], w_ref[...], preferred_element_type=jnp.float32)
    o_ref[...] = r.reshape(1, bm, 8, 128)


def _drive_matmul(x3, w, *, bm, interpret=False):
    b, t, h = x3.shape
    return pl.pallas_call(
        functools.partial(_drive_kernel, bm=bm),
        grid=(b, t // bm),
        in_specs=[pl.BlockSpec((1, bm, h), lambda i, j: (i, j, 0)),
                  pl.BlockSpec((h, MEMORY_DIM), lambda i, j: (0, 0))],
        out_specs=pl.BlockSpec((1, bm, 8, 128), lambda i, j: (i, j, 0, 0)),
        out_shape=jax.ShapeDtypeStruct((b, t, 8, 128), jnp.float32),
        compiler_params=pltpu.CompilerParams(
            dimension_semantics=("arbitrary", "arbitrary")),
        interpret=interpret,
    )(x3, w)


def _out_kernel(n_ref, w_ref, x_ref, alpha_ref, o_ref, *, bm):
    n2 = n_ref[...].reshape(bm, MEMORY_DIM)
    o_ref[0] = (jnp.dot(n2, w_ref[...], preferred_element_type=jnp.float32)
                + alpha_ref[0] * x_ref[0])


def _out_matmul(nre4, w, x3, alpha, *, bm, interpret=False):
    b, t, h = x3.shape
    return pl.pallas_call(
        functools.partial(_out_kernel, bm=bm),
        grid=(b, t // bm),
        in_specs=[pl.BlockSpec((1, bm, 8, 128), lambda i, j: (i, j, 0, 0)),
                  pl.BlockSpec((MEMORY_DIM, h), lambda i, j: (0, 0)),
                  pl.BlockSpec((1, bm, h), lambda i, j: (i, j, 0)),
                  pl.BlockSpec(memory_space=pltpu.SMEM)],
        out_specs=pl.BlockSpec((1, bm, h), lambda i, j: (i, j, 0)),
        out_shape=jax.ShapeDtypeStruct((b, t, h), jnp.float32),
        compiler_params=pltpu.CompilerParams(
            dimension_semantics=("arbitrary", "arbitrary")),
        interpret=interpret,
    )(nre4, w, x3, alpha.reshape(1))


def _scan_kernel(crot_ref, srot_ref, init_re_ref, init_im_ref, drive_ref,
                 nre_ref, tre_s, tim_s, *, tb, nb):
    ch = pl.program_id(0)

    @pl.when(ch == 0)
    def _():
        tre_s[...] = jnp.broadcast_to(init_re_ref[...], (nb, 8, 128))
        tim_s[...] = jnp.broadcast_to(init_im_ref[...], (nb, 8, 128))

    cr = crot_ref[...]
    sr = srot_ref[...]
    cn1, sn1 = cr, sr
    cn2 = cr * cr - sr * sr
    sn2 = 2.0 * cr * sr
    cn3 = cn2 * cr - sn2 * sr
    sn3 = sn2 * cr + cn2 * sr
    g2 = cr * cr + sr * sr
    g4 = g2 * g2
    g6 = g4 * g2
    g8 = g4 * g4
    c1g2 = cn1 * g2
    c1g4 = cn1 * g4
    c2g2 = cn2 * g2

    def rsum(v):
        return jnp.sum(v, axis=(1, 2), keepdims=True)

    def body(i, carry):
        zre, zim = carry
        t0 = 4 * i
        d1 = drive_ref[:, t0]
        d2 = drive_ref[:, t0 + 1]
        d3 = drive_ref[:, t0 + 2]
        d4 = drive_ref[:, t0 + 3]

        d11 = d1 * d1
        d22 = d2 * d2
        d33 = d3 * d3
        d44 = d4 * d4
        d12 = d1 * d2
        d13 = d1 * d3
        d14 = d1 * d4
        d23 = d2 * d3
        d24 = d2 * d4
        d34 = d3 * d4
        G1_11 = rsum(d11)
        G2_11 = rsum(g2 * d11)
        G2_12 = rsum(cn1 * d12)
        G2_22 = rsum(d22)
        G3_11 = rsum(g4 * d11)
        G3_12 = rsum(c1g2 * d12)
        G3_13 = rsum(cn2 * d13)
        G3_22 = rsum(g2 * d22)
        G3_23 = rsum(cn1 * d23)
        G3_33 = rsum(d33)
        G4_11 = rsum(g6 * d11)
        G4_12 = rsum(c1g4 * d12)
        G4_13 = rsum(c2g2 * d13)
        G4_14 = rsum(cn3 * d14)
        G4_22 = rsum(g4 * d22)
        G4_23 = rsum(c1g2 * d23)
        G4_24 = rsum(cn2 * d24)
        G4_33 = rsum(g2 * d33)
        G4_34 = rsum(cn1 * d34)
        G4_44 = rsum(d44)

        z1re = cr * zre - sr * zim
        z1im = sr * zre + cr * zim
        z2re = cr * z1re - sr * z1im
        z2im = sr * z1re + cr * z1im
        z3re = cr * z2re - sr * z2im
        z3im = sr * z2re + cr * z2im
        z4re = cr * z3re - sr * z3im
        z4im = sr * z3re + cr * z3im
        e = zre * zre + zim * zim
        a1 = rsum(g2 * e)
        a2 = rsum(g4 * e)
        a3 = rsum(g6 * e)
        a4 = rsum(g8 * e)
        b11 = rsum(z1re * d1)
        b21 = rsum((z2re * cn1 + z2im * sn1) * d1)
        b22 = rsum(z2re * d2)
        b31 = rsum((z3re * cn2 + z3im * sn2) * d1)
        b32 = rsum((z3re * cn1 + z3im * sn1) * d2)
        b33 = rsum(z3re * d3)
        b41 = rsum((z4re * cn3 + z4im * sn3) * d1)
        b42 = rsum((z4re * cn2 + z4im * sn2) * d2)
        b43 = rsum((z4re * cn1 + z4im * sn1) * d3)
        b44 = rsum(z4re * d4)

        def norm_inv(s):
            return jnp.minimum(jax.lax.rsqrt(jnp.maximum(s, 0.0)), 1e8)

        s1 = a1 + 2.0 * b11 + G1_11
        inv1 = norm_inv(s1)
        sg1 = s1 * inv1
        s2 = (a2 + 2.0 * (b21 + sg1 * b22)
              + G2_11 + 2.0 * sg1 * G2_12 + s1 * G2_22)
        inv2 = norm_inv(s2)
        sg2 = s2 * inv2
        s3 = (a3 + 2.0 * (b31 + sg1 * b32 + sg2 * b33)
              + G3_11 + s1 * G3_22 + s2 * G3_33
              + 2.0 * (sg1 * G3_12 + sg2 * G3_13 + sg1 * sg2 * G3_23))
        inv3 = norm_inv(s3)
        sg3 = s3 * inv3
        s4 = (a4 + 2.0 * (b41 + sg1 * b42 + sg2 * b43 + sg3 * b44)
              + G4_11 + s1 * G4_22 + s2 * G4_33 + s3 * G4_44
              + 2.0 * (sg1 * G4_12 + sg2 * G4_13 + sg3 * G4_14
                       + sg1 * sg2 * G4_23 + sg1 * sg3 * G4_24
                       + sg2 * sg3 * G4_34))
        inv4 = norm_inv(s4)

        nre_ref[:, t0] = (z1re + d1) * inv1
        nre_ref[:, t0 + 1] = (z2re + cn1 * d1 + sg1 * d2) * inv2
        nre_ref[:, t0 + 2] = (z3re + cn2 * d1 + sg1 * (cn1 * d2)
                              + sg2 * d3) * inv3
        w4re = (z4re + cn3 * d1 + sg1 * (cn2 * d2) + sg2 * (cn1 * d3)
                + sg3 * d4)
        w4im = z4im + sn3 * d1 + sg1 * (sn2 * d2) + sg2 * (sn1 * d3)
        zre_n = w4re * inv4
        nre_ref[:, t0 + 3] = zre_n
        return zre_n, w4im * inv4

    tre, tim = jax.lax.fori_loop(0, tb // 4, body,
                                 (tre_s[...], tim_s[...]))
    tre_s[...] = tre
    tim_s[...] = tim


def _run_scan(crot, srot, init_re, init_im, drive4, *, tb, interpret=False):
    b, t = drive4.shape[0], drive4.shape[1]
    nch = t // tb
    grid = (nch,)
    kern = functools.partial(_scan_kernel, tb=tb, nb=b)
    return pl.pallas_call(
        kern,
        grid=grid,
        in_specs=[
            pl.BlockSpec((8, 128), lambda cc: (0, 0)),
            pl.BlockSpec((8, 128), lambda cc: (0, 0)),
            pl.BlockSpec((8, 128), lambda cc: (0, 0)),
            pl.BlockSpec((8, 128), lambda cc: (0, 0)),
            pl.BlockSpec((b, tb, 8, 128), lambda cc: (0, cc, 0, 0)),
        ],
        out_specs=pl.BlockSpec((b, tb, 8, 128), lambda cc: (0, cc, 0, 0)),
        out_shape=jax.ShapeDtypeStruct((b, t, 8, 128), jnp.float32),
        scratch_shapes=[pltpu.VMEM((b, 8, 128), jnp.float32),
                        pltpu.VMEM((b, 8, 128), jnp.float32)],
        compiler_params=pltpu.CompilerParams(
            dimension_semantics=("arbitrary",)),
        interpret=interpret,
    )(crot, srot, init_re, init_im, drive4)


def _kernel_impl(x, tape_init_re, tape_init_im, eta_raw, alpha,
                 epsilon_factor, epsilon_scale, epsilon_diag,
                 pred_factor, pred_scale, pred_diag,
                 torque_rotation, w_r, breadth_gate, basis,
                 interpret=False):
    b, t, h = x.shape
    wa, wc, crot, srot = _fold_weights(
        basis, epsilon_factor, epsilon_scale, epsilon_diag,
        pred_factor, pred_scale, pred_diag,
        breadth_gate, torque_rotation, w_r, eta_raw, interpret=interpret)

    drive4 = _drive_matmul(x, wa, bm=512, interpret=interpret)

    init_re = tape_init_re[:MEMORY_DIM].reshape(8, 128)
    init_im = tape_init_im[:MEMORY_DIM].reshape(8, 128)
    nre4 = _run_scan(crot, srot, init_re, init_im, drive4,
                     tb=256, interpret=interpret)

    return _out_matmul(nre4, wc, x, alpha, bm=512, interpret=interpret)


def kernel(x, tape_init_re, tape_init_im, eta_raw, alpha,
           epsilon_factor, epsilon_scale, epsilon_diag,
           pred_factor, pred_scale, pred_diag,
           torque_rotation, w_r, breadth_gate, basis):
    return _kernel_impl(x, tape_init_re, tape_init_im, eta_raw, alpha,
                        epsilon_factor, epsilon_scale, epsilon_diag,
                        pred_factor, pred_scale, pred_diag,
                        torque_rotation, w_r, breadth_gate, basis)


# speedup vs baseline: 78.9000x; 1.1595x over previous
import functools

import jax
import jax.numpy as jnp
from jax.experimental import pallas as pl
from jax.experimental.pallas import tpu as pltpu

HIDDEN_DIM = 1024
MEMORY_DIM = 1024
TOTAL_SLOTS = 1040
GAMMA = 0.92


def _fused_kernel(x_ref, basis_ref, efac_ref, escale_ref, ediag_ref,
                  pfac_ref, pscale_ref, pdiag_ref, breadth_ref, torque_ref,
                  wr_ref, eta_ref, alpha_ref, init_re_ref, init_im_ref,
                  y_ref,
                  wa_s, wc_s, cr_s, sr_s, zre_s, zim_s, drive_s, nre_s,
                  *, bt, nb):
    c = pl.program_id(0)

    @pl.when(c == 0)
    def _init():
        basis = basis_ref[...]
        efac = efac_ref[...]
        pfac = pfac_ref[...]
        breadth = 1.0 + jnp.tanh(breadth_ref[...])
        eta = jax.nn.softplus(eta_ref[0, 0])

        b1 = basis[:, :MEMORY_DIM]
        br1 = breadth[:, :MEMORY_DIM]
        ed1 = ediag_ref[...][:, :MEMORY_DIM]

        f = jnp.dot(basis, breadth.T * efac,
                    preferred_element_type=jnp.float32)
        low = jnp.dot(f * escale_ref[...],
                      efac[:MEMORY_DIM, :].T,
                      preferred_element_type=jnp.float32)
        wa_s[...] = eta * (b1 * ((1.0 + ed1) * br1) + low)

        bp = jnp.dot(basis, pfac, preferred_element_type=jnp.float32)
        pf1 = pfac[:MEMORY_DIM, :]
        wc_s[...] = (jnp.dot(pf1 * pscale_ref[...], bp.T,
                             preferred_element_type=jnp.float32)
                     + pdiag_ref[...][:, :MEMORY_DIM].T * b1.T)

        leak = jax.nn.sigmoid(wr_ref[...][:, :MEMORY_DIM])
        tq = torque_ref[...][:, :MEMORY_DIM]
        g = GAMMA * leak
        cr_s[...] = (g * jnp.cos(tq)).reshape(8, 128)
        sr_s[...] = (g * jnp.sin(tq)).reshape(8, 128)

        zre_s[...] = jnp.broadcast_to(init_re_ref[...], (nb, 8, 128))
        zim_s[...] = jnp.broadcast_to(init_im_ref[...], (nb, 8, 128))

    x2 = x_ref[...].reshape(nb * bt, HIDDEN_DIM)
    dr = jnp.dot(x2, wa_s[...], preferred_element_type=jnp.float32)
    drive_s[...] = dr.reshape(nb, bt, 8, 128)

    cr = cr_s[...]
    sr = sr_s[...]
    cn1, sn1 = cr, sr
    cn2 = cr * cr - sr * sr
    sn2 = 2.0 * cr * sr
    cn3 = cn2 * cr - sn2 * sr
    sn3 = sn2 * cr + cn2 * sr
    g2 = cr * cr + sr * sr
    g4 = g2 * g2
    g6 = g4 * g2
    g8 = g4 * g4
    c1g2 = cn1 * g2
    c1g4 = cn1 * g4
    c2g2 = cn2 * g2

    def rsum(v):
        return jnp.sum(v, axis=(1, 2), keepdims=True)

    def body(i, carry):
        zre, zim = carry
        t0 = 4 * i
        d1 = drive_s[:, t0]
        d2 = drive_s[:, t0 + 1]
        d3 = drive_s[:, t0 + 2]
        d4 = drive_s[:, t0 + 3]

        d11 = d1 * d1
        d22 = d2 * d2
        d33 = d3 * d3
        d44 = d4 * d4
        d12 = d1 * d2
        d13 = d1 * d3
        d14 = d1 * d4
        d23 = d2 * d3
        d24 = d2 * d4
        d34 = d3 * d4
        G1_11 = rsum(d11)
        G2_11 = rsum(g2 * d11)
        G2_12 = rsum(cn1 * d12)
        G2_22 = rsum(d22)
        G3_11 = rsum(g4 * d11)
        G3_12 = rsum(c1g2 * d12)
        G3_13 = rsum(cn2 * d13)
        G3_22 = rsum(g2 * d22)
        G3_23 = rsum(cn1 * d23)
        G3_33 = rsum(d33)
        G4_11 = rsum(g6 * d11)
        G4_12 = rsum(c1g4 * d12)
        G4_13 = rsum(c2g2 * d13)
        G4_14 = rsum(cn3 * d14)
        G4_22 = rsum(g4 * d22)
        G4_23 = rsum(c1g2 * d23)
        G4_24 = rsum(cn2 * d24)
        G4_33 = rsum(g2 * d33)
        G4_34 = rsum(cn1 * d34)
        G4_44 = rsum(d44)

        z1re = cr * zre - sr * zim
        z1im = sr * zre + cr * zim
        z2re = cr * z1re - sr * z1im
        z2im = sr * z1re + cr * z1im
        z3re = cr * z2re - sr * z2im
        z3im = sr * z2re + cr * z2im
        z4re = cr * z3re - sr * z3im
        z4im = sr * z3re + cr * z3im
        e = zre * zre + zim * zim
        a1 = rsum(g2 * e)
        a2 = rsum(g4 * e)
        a3 = rsum(g6 * e)
        a4 = rsum(g8 * e)
        b11 = rsum(z1re * d1)
        b21 = rsum((z2re * cn1 + z2im * sn1) * d1)
        b22 = rsum(z2re * d2)
        b31 = rsum((z3re * cn2 + z3im * sn2) * d1)
        b32 = rsum((z3re * cn1 + z3im * sn1) * d2)
        b33 = rsum(z3re * d3)
        b41 = rsum((z4re * cn3 + z4im * sn3) * d1)
        b42 = rsum((z4re * cn2 + z4im * sn2) * d2)
        b43 = rsum((z4re * cn1 + z4im * sn1) * d3)
        b44 = rsum(z4re * d4)

        def norm_inv(s):
            return jnp.minimum(jax.lax.rsqrt(jnp.maximum(s, 0.0)), 1e8)

        s1 = a1 + 2.0 * b11 + G1_11
        inv1 = norm_inv(s1)
        sg1 = s1 * inv1
        s2 = (a2 + 2.0 * (b21 + sg1 * b22)
              + G2_11 + 2.0 * sg1 * G2_12 + s1 * G2_22)
        inv2 = norm_inv(s2)
        sg2 = s2 * inv2
        s3 = (a3 + 2.0 * (b31 + sg1 * b32 + sg2 * b33)
              + G3_11 + s1 * G3_22 + s2 * G3_33
              + 2.0 * (sg1 * G3_12 + sg2 * G3_13 + sg1 * sg2 * G3_23))
        inv3 = norm_inv(s3)
        sg3 = s3 * inv3
        s4 = (a4 + 2.0 * (b41 + sg1 * b42 + sg2 * b43 + sg3 * b44)
              + G4_11 + s1 * G4_22 + s2 * G4_33 + s3 * G4_44
              + 2.0 * (sg1 * G4_12 + sg2 * G4_13 + sg3 * G4_14
                       + sg1 * sg2 * G4_23 + sg1 * sg3 * G4_24
                       + sg2 * sg3 * G4_34))
        inv4 = norm_inv(s4)

        nre_s[:, t0] = (z1re + d1) * inv1
        nre_s[:, t0 + 1] = (z2re + cn1 * d1 + sg1 * d2) * inv2
        nre_s[:, t0 + 2] = (z3re + cn2 * d1 + sg1 * (cn1 * d2)
                            + sg2 * d3) * inv3
        w4re = (z4re + cn3 * d1 + sg1 * (cn2 * d2) + sg2 * (cn1 * d3)
                + sg3 * d4)
        w4im = z4im + sn3 * d1 + sg1 * (sn2 * d2) + sg2 * (sn1 * d3)
        zre_n = w4re * inv4
        nre_s[:, t0 + 3] = zre_n
        return zre_n, w4im * inv4

    zre, zim = jax.lax.fori_loop(0, bt // 4, body,
                                 (zre_s[...], zim_s[...]))
    zre_s[...] = zre
    zim_s[...] = zim

    n2 = nre_s[...].reshape(nb * bt, MEMORY_DIM)
    y = (jnp.dot(n2, wc_s[...], preferred_element_type=jnp.float32)
         + alpha_ref[0] * x2)
    y_ref[...] = y.reshape(nb, bt, HIDDEN_DIM)


def _kernel_impl(x, tape_init_re, tape_init_im, eta_raw, alpha,
                 epsilon_factor, epsilon_scale, epsilon_diag,
                 pred_factor, pred_scale, pred_diag,
                 torque_rotation, w_r, breadth_gate, basis,
                 interpret=False):
    b, t, h = x.shape
    bt = 512
    nch = t // bt
    init_re = tape_init_re[:MEMORY_DIM].reshape(8, 128)
    init_im = tape_init_im[:MEMORY_DIM].reshape(8, 128)
    full = lambda cc: tuple(0 for _ in range(2))

    kern = functools.partial(_fused_kernel, bt=bt, nb=b)
    return pl.pallas_call(
        kern,
        grid=(nch,),
        in_specs=[
            pl.BlockSpec((b, bt, h), lambda cc: (0, cc, 0)),
            pl.BlockSpec((h, TOTAL_SLOTS), full),
            pl.BlockSpec((TOTAL_SLOTS, 10), full),
            pl.BlockSpec((1, 10), full),
            pl.BlockSpec((1, TOTAL_SLOTS), full),
            pl.BlockSpec((TOTAL_SLOTS, 10), full),
            pl.BlockSpec((1, 10), full),
            pl.BlockSpec((1, TOTAL_SLOTS), full),
            pl.BlockSpec((1, TOTAL_SLOTS), full),
            pl.BlockSpec((1, TOTAL_SLOTS), full),
            pl.BlockSpec((1, TOTAL_SLOTS), full),
            pl.BlockSpec((1, 1), full),
            pl.BlockSpec(memory_space=pltpu.SMEM),
            pl.BlockSpec((8, 128), full),
            pl.BlockSpec((8, 128), full),
        ],
        out_specs=pl.BlockSpec((b, bt, h), lambda cc: (0, cc, 0)),
        out_shape=jax.ShapeDtypeStruct((b, t, h), jnp.float32),
        scratch_shapes=[
            pltpu.VMEM((MEMORY_DIM, MEMORY_DIM), jnp.float32),
            pltpu.VMEM((MEMORY_DIM, MEMORY_DIM), jnp.float32),
            pltpu.VMEM((8, 128), jnp.float32),
            pltpu.VMEM((8, 128), jnp.float32),
            pltpu.VMEM((b, 8, 128), jnp.float32),
            pltpu.VMEM((b, 8, 128), jnp.float32),
            pltpu.VMEM((b, bt, 8, 128), jnp.float32),
            pltpu.VMEM((b, bt, 8, 128), jnp.float32),
        ],
        compiler_params=pltpu.CompilerParams(
            dimension_semantics=("arbitrary",)),
        interpret=interpret,
    )(x, basis, epsilon_factor, epsilon_scale.reshape(1, -1),
      epsilon_diag.reshape(1, -1), pred_factor, pred_scale.reshape(1, -1),
      pred_diag.reshape(1, -1), breadth_gate.reshape(1, -1),
      torque_rotation.reshape(1, -1), w_r.reshape(1, -1),
      eta_raw.reshape(1, 1), alpha.reshape(1), init_re, init_im)


def kernel(x, tape_init_re, tape_init_im, eta_raw, alpha,
           epsilon_factor, epsilon_scale, epsilon_diag,
           pred_factor, pred_scale, pred_diag,
           torque_rotation, w_r, breadth_gate, basis):
    return _kernel_impl(x, tape_init_re, tape_init_im, eta_raw, alpha,
                        epsilon_factor, epsilon_scale, epsilon_diag,
                        pred_factor, pred_scale, pred_diag,
                        torque_rotation, w_r, breadth_gate, basis)
